# Initial kernel scaffold; baseline (speedup 1.0000x reference)
import jax
import jax.numpy as jnp
from jax.experimental import pallas as pl


def kernel(x, conv_w, conv_b, gamma, bn_beta, lif_beta):
    raise NotImplementedError("write your pallas kernel here")


# speedup vs baseline: 20.7870x; 20.7870x over previous
import jax
import jax.numpy as jnp
from jax.experimental import pallas as pl
from jax.experimental.pallas import tpu as pltpu

_B, _T, _C = 32, 512, 512
_H, _S = 512, 4
_N = _B * _T
_THR = 1.0
_EPS = 1e-5

_RC = 1024
_NRC = _N // _RC
_TC = 32
_NTC = _T // _TC
_BH = _B // 2


def _gemm_body(xt_ref, w_ref, cb_ref, h_ref, st_ref, acc_ref):
    rc = pl.program_id(1)
    h = jax.lax.dot_general(
        xt_ref[...], w_ref[...], (((1,), (1,)), ((), ())),
        preferred_element_type=jnp.float32)
    h = h + cb_ref[0]
    h_ref[0] = h

    @pl.when(rc == 0)
    def _():
        acc_ref[...] = jnp.zeros_like(acc_ref)

    acc_ref[0:1] += jnp.sum(h, axis=0, keepdims=True)
    acc_ref[1:2] += jnp.sum(h * h, axis=0, keepdims=True)

    @pl.when(rc == _NRC - 1)
    def _():
        st_ref[0] = acc_ref[...]


def _scan_body(h_ref, st_ref, g_ref, bb_ref, beta_ref,
               spk_ref, mem_ref, cnt_ref, mem_s, acc_s):
    tc = pl.program_id(1)
    beta = beta_ref[0, 0]
    inv_n = jnp.float32(1.0 / _N)
    mean = st_ref[0] * inv_n
    var = st_ref[1] * inv_n - mean * mean
    rs = jax.lax.rsqrt(var + _EPS)
    g = g_ref[...]
    bb = bb_ref[...]

    @pl.when(tc == 0)
    def _():
        mem_s[...] = jnp.zeros_like(mem_s)
        acc_s[...] = jnp.zeros_like(acc_s)

    def body(tt, carry):
        mem, acc = carry
        for s in range(_S):
            h = h_ref[s, tt]
            hb = ((h - mean[s]) * rs[s]) * g[s] + bb[s]
            reset = (mem > _THR).astype(jnp.float32)
            mem = beta * mem + hb - reset * _THR
            spk = (mem > _THR).astype(jnp.float32)
            spk_ref[tt, s] = spk
            mem_ref[tt, s] = mem
            acc = acc + spk
        return (mem, acc)

    mem1, acc1 = jax.lax.fori_loop(0, _TC, body, (mem_s[...], acc_s[...]))
    mem_s[...] = mem1
    acc_s[...] = acc1

    @pl.when(tc == _NTC - 1)
    def _():
        cnt_ref[0] = acc1


def kernel(x, conv_w, conv_b, gamma, bn_beta, lif_beta):
    xt = x.transpose(1, 0, 2).reshape(_N, _C)
    cb3 = conv_b.reshape(_S, 1, _H)

    h, stats = pl.pallas_call(
        _gemm_body,
        grid=(_S, _NRC),
        in_specs=[
            pl.BlockSpec((_RC, _C), lambda s, r: (r, 0)),
            pl.BlockSpec((_H, _C), lambda s, r: (s, 0)),
            pl.BlockSpec((1, 1, _H), lambda s, r: (s, 0, 0)),
        ],
        out_specs=[
            pl.BlockSpec((1, _RC, _H), lambda s, r: (s, r, 0)),
            pl.BlockSpec((1, 2, _H), lambda s, r: (s, 0, 0)),
        ],
        out_shape=[
            jax.ShapeDtypeStruct((_S, _N, _H), jnp.float32),
            jax.ShapeDtypeStruct((_S, 2, _H), jnp.float32),
        ],
        scratch_shapes=[pltpu.VMEM((2, _H), jnp.float32)],
        compiler_params=pltpu.CompilerParams(
            dimension_semantics=("parallel", "arbitrary")),
        name="conv_bn_gemm",
    )(xt, conv_w, cb3)

    h4 = h.reshape(_S, _T, _B, _H)
    stats2 = stats.transpose(1, 0, 2)
    g2 = gamma.reshape(_S, _H)
    bb2 = bn_beta.reshape(_S, _H)
    beta2 = jnp.reshape(lif_beta, (1, 1))

    spk4, mem4, cnt = pl.pallas_call(
        _scan_body,
        grid=(2, _NTC),
        in_specs=[
            pl.BlockSpec((_S, _TC, _BH, _H), lambda b, t: (0, t, b, 0)),
            pl.BlockSpec((2, _S, _H), lambda b, t: (0, 0, 0)),
            pl.BlockSpec((_S, _H), lambda b, t: (0, 0)),
            pl.BlockSpec((_S, _H), lambda b, t: (0, 0)),
            pl.BlockSpec(memory_space=pltpu.SMEM),
        ],
        out_specs=[
            pl.BlockSpec((_TC, _S, _BH, _H), lambda b, t: (t, 0, b, 0)),
            pl.BlockSpec((_TC, _S, _BH, _H), lambda b, t: (t, 0, b, 0)),
            pl.BlockSpec((1, _BH, _H), lambda b, t: (b, 0, 0)),
        ],
        out_shape=[
            jax.ShapeDtypeStruct((_T, _S, _B, _H), jnp.float32),
            jax.ShapeDtypeStruct((_T, _S, _B, _H), jnp.float32),
            jax.ShapeDtypeStruct((2, _BH, _H), jnp.float32),
        ],
        scratch_shapes=[pltpu.VMEM((_BH, _H), jnp.float32),
                        pltpu.VMEM((_BH, _H), jnp.float32)],
        compiler_params=pltpu.CompilerParams(
            dimension_semantics=("parallel", "arbitrary")),
        name="lif_scan",
    )(h4, stats2, g2, bb2, beta2)

    spk_rec = spk4.reshape(_T * _S, _B, _H)
    mem_rec = mem4.reshape(_T * _S, _B, _H)
    sum_spks = cnt.sum()
    return spk_rec, mem_rec, sum_spks


# speedup vs baseline: 28.7163x; 1.3815x over previous
import jax
import jax.numpy as jnp
from jax.experimental import pallas as pl
from jax.experimental.pallas import tpu as pltpu

_B, _T, _C = 32, 512, 512
_H, _S = 512, 4
_OUT = _H * _S
_N = _B * _T
_THR = 1.0
_EPS = 1e-5

_RC = 1024
_NRC = _N // _RC
_TC = 32
_NTC = _T // _TC
_HH = _H // 2


def _stats_body(xt_ref, w_ref, cb_ref, st_ref):
    for s in range(_S):
        h = jax.lax.dot_general(
            xt_ref[...], w_ref[s * _H:(s + 1) * _H, :],
            (((1,), (1,)), ((), ())),
            preferred_element_type=jnp.float32) + cb_ref[s]
        st_ref[0, s] = jnp.concatenate(
            [jnp.sum(h, axis=0, keepdims=True),
             jnp.sum(h * h, axis=0, keepdims=True)], axis=0)


def _scan_body(xt_ref, w_ref, cb_ref, st_ref, g_ref, bb_ref, beta_ref,
               spk_ref, mem_ref, cnt_ref, hbuf, mem_s, acc_s):
    tc = pl.program_id(1)
    cb = cb_ref[...]
    for s in range(_S):
        hbuf[s] = jax.lax.dot_general(
            xt_ref[...], w_ref[s], (((1,), (1,)), ((), ())),
            preferred_element_type=jnp.float32) + cb[s]

    beta = beta_ref[0, 0]
    inv_n = jnp.float32(1.0 / _N)
    sums = st_ref[:, 0]
    mean = sums * inv_n
    var = st_ref[:, 1] * inv_n - mean * mean
    rs = jax.lax.rsqrt(var + _EPS)
    g = g_ref[...]
    bb = bb_ref[...]

    @pl.when(tc == 0)
    def _():
        mem_s[...] = jnp.zeros_like(mem_s)
        acc_s[...] = jnp.zeros_like(acc_s)

    def body(tt, carry):
        mem, acc = carry
        for s in range(_S):
            h = hbuf[s, pl.ds(tt * _B, _B), :]
            hb = ((h - mean[s]) * rs[s]) * g[s] + bb[s]
            reset = (mem > _THR).astype(jnp.float32)
            mem = beta * mem + hb - reset * _THR
            spk = (mem > _THR).astype(jnp.float32)
            spk_ref[tt * _S + s] = spk
            mem_ref[tt * _S + s] = mem
            acc = acc + spk
        return (mem, acc)

    mem1, acc1 = jax.lax.fori_loop(0, _TC, body, (mem_s[...], acc_s[...]))
    mem_s[...] = mem1
    acc_s[...] = acc1

    @pl.when(tc == _NTC - 1)
    def _():
        cnt_ref[0] = acc1


def kernel(x, conv_w, conv_b, gamma, bn_beta, lif_beta):
    xt = x.transpose(1, 0, 2).reshape(_N, _C)

    parts = pl.pallas_call(
        _stats_body,
        grid=(_NRC,),
        in_specs=[
            pl.BlockSpec((_RC, _C), lambda r: (r, 0)),
            pl.BlockSpec((_OUT, _C), lambda r: (0, 0)),
            pl.BlockSpec((_S, _H), lambda r: (0, 0)),
        ],
        out_specs=pl.BlockSpec((1, _S, 2, _H), lambda r: (r, 0, 0, 0)),
        out_shape=jax.ShapeDtypeStruct((_NRC, _S, 2, _H), jnp.float32),
        compiler_params=pltpu.CompilerParams(
            dimension_semantics=("parallel",)),
        name="bn_stats",
    )(xt, conv_w, conv_b.reshape(_S, _H))

    stats = parts.sum(axis=0)
    w4 = conv_w.reshape(_S, _H, _C)
    cb4 = conv_b.reshape(_S, _H)
    g4 = gamma.reshape(_S, _H)
    bb4 = bn_beta.reshape(_S, _H)
    beta2 = jnp.reshape(lif_beta, (1, 1))

    spk_rec, mem_rec, cnt = pl.pallas_call(
        _scan_body,
        grid=(2, _NTC),
        in_specs=[
            pl.BlockSpec((_TC * _B, _C), lambda hh, t: (t, 0)),
            pl.BlockSpec((_S, _HH, _C), lambda hh, t: (0, hh, 0)),
            pl.BlockSpec((_S, _HH), lambda hh, t: (0, hh)),
            pl.BlockSpec((_S, 2, _HH), lambda hh, t: (0, 0, hh)),
            pl.BlockSpec((_S, _HH), lambda hh, t: (0, hh)),
            pl.BlockSpec((_S, _HH), lambda hh, t: (0, hh)),
            pl.BlockSpec(memory_space=pltpu.SMEM),
        ],
        out_specs=[
            pl.BlockSpec((_TC * _S, _B, _HH), lambda hh, t: (t, 0, hh)),
            pl.BlockSpec((_TC * _S, _B, _HH), lambda hh, t: (t, 0, hh)),
            pl.BlockSpec((1, _B, _HH), lambda hh, t: (hh, 0, 0)),
        ],
        out_shape=[
            jax.ShapeDtypeStruct((_T * _S, _B, _H), jnp.float32),
            jax.ShapeDtypeStruct((_T * _S, _B, _H), jnp.float32),
            jax.ShapeDtypeStruct((2, _B, _HH), jnp.float32),
        ],
        scratch_shapes=[pltpu.VMEM((_S, _TC * _B, _HH), jnp.float32),
                        pltpu.VMEM((_B, _HH), jnp.float32),
                        pltpu.VMEM((_B, _HH), jnp.float32)],
        compiler_params=pltpu.CompilerParams(
            dimension_semantics=("parallel", "arbitrary"),
            vmem_limit_bytes=50 * 1024 * 1024),
        name="lif_gemm_scan",
    )(xt, w4, cb4, stats, g4, bb4, beta2)

    sum_spks = cnt.sum()
    return spk_rec, mem_rec, sum_spks
